# Initial kernel scaffold; baseline (speedup 1.0000x reference)
import jax
import jax.numpy as jnp
from jax.experimental import pallas as pl


def kernel(rays_origins, rays_directions, rays_lengths, rays_features):
    raise NotImplementedError("write your pallas kernel here")


# speedup vs baseline: 17.3457x; 17.3457x over previous
import jax
import jax.numpy as jnp
from jax import lax
from jax.experimental import pallas as pl
from jax.experimental.pallas import tpu as pltpu, tpu_sc as plsc

_B, _R, _P, _C = 2, 16384, 64, 4
_D, _Hh, _W = 64, 64, 64
_DHW = _D * _Hh * _W
_NVOL = _DHW * _C

_NS = 16
_RT = _R // _NS
_RC = 128
_NSUB = _RT // _RC
_NPW = _P // 16
_NE = 512
_SLICE = _NVOL // _NS


def _body(geo_hbm, len_hbm, feat_hbm, out_hbm,
          vol_sh, geo_v, len_v, feat_v, idx_c, idx_e, val_e, zb,
          wx0, wx1, wy0, wy1, wz0, wz1):
    c = lax.axis_index("c")
    s = lax.axis_index("s")
    iota = lax.iota(jnp.int32, 16)
    row_e = iota >> 2
    col_e = iota & 3
    fzero = jnp.zeros((16,), jnp.float32)

    def zb_init(i, carry):
        zb[pl.ds(i * 16, 16)] = fzero
        return carry
    lax.fori_loop(0, 4096 // 16, zb_init, 0)
    base = s * _SLICE

    def zdma(j, carry):
        pltpu.sync_copy(zb, vol_sh.at[pl.ds(base + j * 4096, 4096)])
        return carry
    lax.fori_loop(0, _SLICE // 4096, zdma, 0)
    plsc.subcore_barrier()

    r0_tile = s * _RT

    def sub_body(sub, carry):
        r0 = r0_tile + sub * _RC
        pltpu.sync_copy(geo_hbm.at[c, pl.ds(r0 * 6, _RC * 6)],
                        geo_v.at[pl.ds(0, _RC * 6)])
        pltpu.sync_copy(len_hbm.at[c, pl.ds(r0, _RC)], len_v)
        pltpu.sync_copy(feat_hbm.at[c, pl.ds(r0, _RC)], feat_v)

        def ray_body(r, carry2):
            g = geo_v[pl.ds(r * 6, 16)]
            ox = g[0] * 32.0 + 31.5
            oy = g[1] * 32.0 + 31.5
            oz = g[2] * 32.0 + 31.5
            dx = g[3] * 32.0
            dy = g[4] * 32.0
            dz = g[5] * 32.0
            oxv = jnp.full((16,), ox, jnp.float32)
            oyv = jnp.full((16,), oy, jnp.float32)
            ozv = jnp.full((16,), oz, jnp.float32)
            dxv = jnp.full((16,), dx, jnp.float32)
            dyv = jnp.full((16,), dy, jnp.float32)
            dzv = jnp.full((16,), dz, jnp.float32)

            def pw_body(pw, carry3):
                t = len_v[r, pl.ds(pw * 16, 16)]

                def axis(ov, dv, scale):
                    v = ov + dv * t
                    i0 = v.astype(jnp.int32)
                    f0 = i0.astype(jnp.float32)
                    bi = i0 - jnp.where(f0 > v, 1, 0)
                    fr = v - bi.astype(jnp.float32)
                    va0 = (bi >= 0) & (bi <= 63)
                    va1 = (bi >= -1) & (bi <= 62)
                    w0 = jnp.where(va0, 1.0 - fr, 0.0)
                    w1 = jnp.where(va1, fr, 0.0)
                    c0 = jnp.minimum(jnp.maximum(bi, 0), 63) * scale
                    c1 = jnp.minimum(jnp.maximum(bi + 1, 0), 63) * scale
                    return w0, w1, c0, c1

                x0w, x1w, ix0, ix1 = axis(oxv, dxv, 1)
                y0w, y1w, iy0, iy1 = axis(oyv, dyv, 64)
                z0w, z1w, iz0, iz1 = axis(ozv, dzv, 4096)
                wx0[...] = x0w
                wx1[...] = x1w
                wy0[...] = y0w
                wy1[...] = y1w
                wz0[...] = z0w
                wz1[...] = z1w

                izy = (iz0 + iy0, iz0 + iy1, iz1 + iy0, iz1 + iy1)
                k = 0
                for a in range(4):
                    for dxi in (ix0, ix1):
                        idx_c[pl.ds(k * 16, 16)] = izy[a] + dxi
                        k += 1

                for q in range(4):
                    e = row_e + q * 4
                    gx0 = plsc.load_gather(wx0, [e])
                    gx1 = plsc.load_gather(wx1, [e])
                    gy0 = plsc.load_gather(wy0, [e])
                    gy1 = plsc.load_gather(wy1, [e])
                    gz0 = plsc.load_gather(wz0, [e])
                    gz1 = plsc.load_gather(wz1, [e])
                    f16 = feat_v[r, pl.ds(pw * 64 + q * 16, 16)]
                    wzy = (gz0 * gy0, gz0 * gy1, gz1 * gy0, gz1 * gy1)
                    k = 0
                    for a in range(4):
                        for gx in (gx0, gx1):
                            off = k * 64 + q * 16
                            gi = plsc.load_gather(idx_c, [e + k * 16])
                            idx_e[pl.ds(off, 16)] = (gi << 2) | col_e
                            val_e[pl.ds(off, 16)] = f16 * (wzy[a] * gx)
                            k += 1

                pltpu.sync_copy(val_e, vol_sh.at[idx_e], add=True)
                return carry3

            lax.fori_loop(0, _NPW, pw_body, 0)
            return carry2

        lax.fori_loop(0, _RC, ray_body, 0)
        return carry

    lax.fori_loop(0, _NSUB, sub_body, 0)

    plsc.subcore_barrier()
    pltpu.sync_copy(vol_sh.at[pl.ds(base, _SLICE)],
                    out_hbm.at[c, pl.ds(base, _SLICE)])


_sc_call = pl.kernel(
    _body,
    out_type=jax.ShapeDtypeStruct((_B, _NVOL), jnp.float32),
    mesh=plsc.VectorSubcoreMesh(core_axis_name="c", subcore_axis_name="s"),
    compiler_params=pltpu.CompilerParams(
        needs_layout_passes=False, use_tc_tiling_on_sc=False),
    scratch_types=[
        pltpu.VMEM_SHARED((_NVOL,), jnp.float32),
        pltpu.VMEM((_RC * 6 + 16,), jnp.float32),
        pltpu.VMEM((_RC, _P), jnp.float32),
        pltpu.VMEM((_RC, _P * _C), jnp.float32),
        pltpu.VMEM((128,), jnp.int32),
        pltpu.VMEM((_NE,), jnp.int32),
        pltpu.VMEM((_NE,), jnp.float32),
        pltpu.VMEM((4096,), jnp.float32),
        pltpu.VMEM((16,), jnp.float32),
        pltpu.VMEM((16,), jnp.float32),
        pltpu.VMEM((16,), jnp.float32),
        pltpu.VMEM((16,), jnp.float32),
        pltpu.VMEM((16,), jnp.float32),
        pltpu.VMEM((16,), jnp.float32),
    ],
)


@jax.jit
def kernel(rays_origins, rays_directions, rays_lengths, rays_features):
    geo = jnp.concatenate(
        [rays_origins, rays_directions], axis=-1).reshape(_B, _R * 6)
    ft = rays_features.reshape(_B, _R, _P * _C)
    vol = _sc_call(geo, rays_lengths, ft)
    return vol.reshape(_B, _D, _Hh, _W, _C).transpose(0, 4, 1, 2, 3)


# speedup vs baseline: 18.2889x; 1.0544x over previous
import jax
import jax.numpy as jnp
from jax import lax
from jax.experimental import pallas as pl
from jax.experimental.pallas import tpu as pltpu, tpu_sc as plsc

_B, _R, _P, _C = 2, 16384, 64, 4
_D, _Hh, _W = 64, 64, 64
_DHW = _D * _Hh * _W
_NVOL = _DHW * _C

_NS = 16
_RT = _R // _NS
_RC = 128
_NSUB = _RT // _RC
_NPW = _P // 16
_NE = 2048
_SLICE = _NVOL // _NS


def _body(geo_hbm, len_hbm, feat_hbm, out_hbm,
          vol_sh, geo_v, len_v, feat_v, idx_c, idx_a, val_a, idx_b, val_b,
          zb, wx0, wx1, wy0, wy1, wz0, wz1, sem_a, sem_b):
    c = lax.axis_index("c")
    s = lax.axis_index("s")
    iota = lax.iota(jnp.int32, 16)
    row_e = iota >> 2
    col_e = iota & 3
    fzero = jnp.zeros((16,), jnp.float32)

    def zb_init(i, carry):
        zb[pl.ds(i * 16, 16)] = fzero
        return carry
    lax.fori_loop(0, 4096 // 16, zb_init, 0)
    base = s * _SLICE

    def zdma(j, carry):
        pltpu.sync_copy(zb, vol_sh.at[pl.ds(base + j * 4096, 4096)])
        return carry
    lax.fori_loop(0, _SLICE // 4096, zdma, 0)
    plsc.subcore_barrier()

    r0_tile = s * _RT

    def sub_body(sub, carry):
        r0 = r0_tile + sub * _RC
        pltpu.sync_copy(geo_hbm.at[c, pl.ds(r0 * 6, _RC * 6)],
                        geo_v.at[pl.ds(0, _RC * 6)])
        pltpu.sync_copy(len_hbm.at[c, pl.ds(r0, _RC)], len_v)
        pltpu.sync_copy(feat_hbm.at[c, pl.ds(r0, _RC)], feat_v)

        def pair_body(rp, carry2):
            for half, (idx_e, val_e, sem) in enumerate(
                    ((idx_a, val_a, sem_a), (idx_b, val_b, sem_b))):
                r = rp * 2 + half
                @pl.when((sub > 0) | (rp > 0))
                def _():
                    pltpu.make_async_copy(
                        val_e, vol_sh.at[idx_e], sem).wait()
                _fill_ray(r, idx_e, val_e)
                pltpu.async_copy(val_e, vol_sh.at[idx_e], sem, add=True)
            return carry2

        def _fill_ray(r, idx_e, val_e):
            g = geo_v[pl.ds(r * 6, 16)]
            ox = g[0] * 32.0 + 31.5
            oy = g[1] * 32.0 + 31.5
            oz = g[2] * 32.0 + 31.5
            dx = g[3] * 32.0
            dy = g[4] * 32.0
            dz = g[5] * 32.0
            oxv = jnp.full((16,), ox, jnp.float32)
            oyv = jnp.full((16,), oy, jnp.float32)
            ozv = jnp.full((16,), oz, jnp.float32)
            dxv = jnp.full((16,), dx, jnp.float32)
            dyv = jnp.full((16,), dy, jnp.float32)
            dzv = jnp.full((16,), dz, jnp.float32)

            def pw_body(pw, carry3):
                t = len_v[r, pl.ds(pw * 16, 16)]

                def axis(ov, dv, scale):
                    v = ov + dv * t
                    i0 = v.astype(jnp.int32)
                    f0 = i0.astype(jnp.float32)
                    bi = i0 - jnp.where(f0 > v, 1, 0)
                    fr = v - bi.astype(jnp.float32)
                    va0 = (bi >= 0) & (bi <= 63)
                    va1 = (bi >= -1) & (bi <= 62)
                    w0 = jnp.where(va0, 1.0 - fr, 0.0)
                    w1 = jnp.where(va1, fr, 0.0)
                    c0 = jnp.minimum(jnp.maximum(bi, 0), 63) * scale
                    c1 = jnp.minimum(jnp.maximum(bi + 1, 0), 63) * scale
                    return w0, w1, c0, c1

                x0w, x1w, ix0, ix1 = axis(oxv, dxv, 1)
                y0w, y1w, iy0, iy1 = axis(oyv, dyv, 64)
                z0w, z1w, iz0, iz1 = axis(ozv, dzv, 4096)
                wx0[...] = x0w
                wx1[...] = x1w
                wy0[...] = y0w
                wy1[...] = y1w
                wz0[...] = z0w
                wz1[...] = z1w

                izy = (iz0 + iy0, iz0 + iy1, iz1 + iy0, iz1 + iy1)
                k = 0
                for a in range(4):
                    for dxi in (ix0, ix1):
                        idx_c[pl.ds(k * 16, 16)] = izy[a] + dxi
                        k += 1

                for q in range(4):
                    e = row_e + q * 4
                    gx0 = plsc.load_gather(wx0, [e])
                    gx1 = plsc.load_gather(wx1, [e])
                    gy0 = plsc.load_gather(wy0, [e])
                    gy1 = plsc.load_gather(wy1, [e])
                    gz0 = plsc.load_gather(wz0, [e])
                    gz1 = plsc.load_gather(wz1, [e])
                    f16 = feat_v[r, pl.ds(pw * 64 + q * 16, 16)]
                    wzy = (gz0 * gy0, gz0 * gy1, gz1 * gy0, gz1 * gy1)
                    k = 0
                    for a in range(4):
                        for gx in (gx0, gx1):
                            off = pw * 512 + k * 64 + q * 16
                            gi = plsc.load_gather(idx_c, [e + k * 16])
                            idx_e[pl.ds(off, 16)] = (gi << 2) | col_e
                            val_e[pl.ds(off, 16)] = f16 * (wzy[a] * gx)
                            k += 1
                return carry3

            lax.fori_loop(0, _NPW, pw_body, 0)

        lax.fori_loop(0, _RC // 2, pair_body, 0)
        return carry

    lax.fori_loop(0, _NSUB, sub_body, 0)

    pltpu.make_async_copy(val_a, vol_sh.at[idx_a], sem_a).wait()
    pltpu.make_async_copy(val_b, vol_sh.at[idx_b], sem_b).wait()
    plsc.subcore_barrier()
    pltpu.sync_copy(vol_sh.at[pl.ds(base, _SLICE)],
                    out_hbm.at[c, pl.ds(base, _SLICE)])


_sc_call = pl.kernel(
    _body,
    out_type=jax.ShapeDtypeStruct((_B, _NVOL), jnp.float32),
    mesh=plsc.VectorSubcoreMesh(core_axis_name="c", subcore_axis_name="s"),
    compiler_params=pltpu.CompilerParams(
        needs_layout_passes=False, use_tc_tiling_on_sc=False),
    scratch_types=[
        pltpu.VMEM_SHARED((_NVOL,), jnp.float32),
        pltpu.VMEM((_RC * 6 + 16,), jnp.float32),
        pltpu.VMEM((_RC, _P), jnp.float32),
        pltpu.VMEM((_RC, _P * _C), jnp.float32),
        pltpu.VMEM((128,), jnp.int32),
        pltpu.VMEM((_NE,), jnp.int32),
        pltpu.VMEM((_NE,), jnp.float32),
        pltpu.VMEM((_NE,), jnp.int32),
        pltpu.VMEM((_NE,), jnp.float32),
        pltpu.VMEM((4096,), jnp.float32),
        pltpu.VMEM((16,), jnp.float32),
        pltpu.VMEM((16,), jnp.float32),
        pltpu.VMEM((16,), jnp.float32),
        pltpu.VMEM((16,), jnp.float32),
        pltpu.VMEM((16,), jnp.float32),
        pltpu.VMEM((16,), jnp.float32),
        pltpu.SemaphoreType.DMA,
        pltpu.SemaphoreType.DMA,
    ],
)


@jax.jit
def kernel(rays_origins, rays_directions, rays_lengths, rays_features):
    geo = jnp.concatenate(
        [rays_origins, rays_directions], axis=-1).reshape(_B, _R * 6)
    ft = rays_features.reshape(_B, _R, _P * _C)
    vol = _sc_call(geo, rays_lengths, ft)
    return vol.reshape(_B, _D, _Hh, _W, _C).transpose(0, 4, 1, 2, 3)
